# baseline (speedup 1.0000x reference)
import jax
import jax.numpy as jnp
from jax.experimental import pallas as pl
from jax.experimental.pallas import tpu as pltpu

_SCALE = 0.5
_EPS = 1e-05

_B = 32
_H = 512
_W = 512
_K = 4
_NSTEPS = _B // _K


def _berhu_body(p0, p1, p2, p3, g0, g1, g2, g3, out_ref, dv_ref,
                w_ref, b_ref, acc_ref):
    i = pl.program_id(0)

    @pl.when(i == 0)
    def _init():
        acc_ref[0] = 0.0
        acc_ref[1] = 0.0
        acc_ref[2] = 0.0

    s = 0.0
    m = 0.0
    c = 0.0
    for k, (pr, gr) in enumerate(((p0, g0), (p1, g1), (p2, g2), (p3, g3))):
        p = pr[0, 0]
        g = gr[0, 0]
        valid = g > _EPS
        dv = jnp.where(valid, jnp.abs(p - g), 0.0)
        dv_ref[_K * i + k] = dv
        s = s + jnp.sum(dv)
        m = jnp.maximum(m, jnp.max(dv))
        c = c + jnp.sum(jnp.where(valid, 1.0, 0.0))
    acc_ref[0] = jnp.maximum(acc_ref[0], m)
    acc_ref[1] = acc_ref[1] + s
    acc_ref[2] = acc_ref[2] + c

    @pl.when(i == _NSTEPS - 1)
    def _finish():
        t = _SCALE * acc_ref[0]
        denom = 2.0 * t + _EPS
        w_ref[...] = jnp.zeros_like(w_ref)
        b_ref[...] = jnp.zeros_like(b_ref)

        def loop(j, _):
            blk = dv_ref[j]
            q = jnp.maximum(blk - t, 0.0)
            w_ref[...] = w_ref[...] + q * q
            b_ref[...] = b_ref[...] + jnp.where(blk > t, blk, 0.0)
            return 0

        jax.lax.fori_loop(0, _B, loop, 0)
        total = acc_ref[1] + (
            jnp.sum(w_ref[...]) - _EPS * jnp.sum(b_ref[...])) / denom
        out_ref[0] = total / acc_ref[2]


def kernel(pred, gt):
    def spec(k):
        return pl.BlockSpec((1, 1, _H, _W), lambda i, k=k: (_K * i + k, 0, 0, 0))

    out = pl.pallas_call(
        _berhu_body,
        grid=(_NSTEPS,),
        in_specs=[spec(k) for k in range(_K)] * 2,
        out_specs=pl.BlockSpec(memory_space=pltpu.SMEM),
        out_shape=jax.ShapeDtypeStruct((1,), jnp.float32),
        scratch_shapes=[
            pltpu.VMEM((_B, _H, _W), jnp.float32),
            pltpu.VMEM((_H, _W), jnp.float32),
            pltpu.VMEM((_H, _W), jnp.float32),
            pltpu.SMEM((4,), jnp.float32),
        ],
        compiler_params=pltpu.CompilerParams(
            vmem_limit_bytes=58 * 1024 * 1024,
        ),
    )(pred, pred, pred, pred, gt, gt, gt, gt)
    return out[0]


# speedup vs baseline: 1.1598x; 1.1598x over previous
import jax
import jax.numpy as jnp
from jax.experimental import pallas as pl
from jax.experimental.pallas import tpu as pltpu

_SCALE = 0.5
_EPS = 1e-05

_B = 32
_H = 512
_W = 512
_K = 4
_NSTEPS = _B // _K


def _berhu_body(p0, p1, p2, p3, g0, g1, g2, g3, out_ref, dv_ref,
                s_ref, m_ref, c_ref, w_ref):
    i = pl.program_id(0)

    @pl.when(i == 0)
    def _init():
        s_ref[...] = jnp.zeros_like(s_ref)
        m_ref[...] = jnp.zeros_like(m_ref)
        c_ref[...] = jnp.zeros_like(c_ref)

    s = s_ref[...]
    m = m_ref[...]
    c = c_ref[...]
    for k, (pr, gr) in enumerate(((p0, g0), (p1, g1), (p2, g2), (p3, g3))):
        p = pr[0, 0]
        g = gr[0, 0]
        valid = g > _EPS
        dv = jnp.where(valid, jnp.abs(p - g), 0.0)
        dv_ref[_K * i + k] = dv
        s = s + dv
        m = jnp.maximum(m, dv)
        c = c + jnp.where(valid, 1.0, 0.0)
    s_ref[...] = s
    m_ref[...] = m
    c_ref[...] = c

    @pl.when(i == _NSTEPS - 1)
    def _finish():
        t = _SCALE * jnp.max(m_ref[...])
        denom = 2.0 * t + _EPS
        w_ref[...] = jnp.zeros_like(w_ref)

        def loop(j, _):
            q = jnp.maximum(dv_ref[j] - t, 0.0)
            w_ref[...] = w_ref[...] + q * q
            return 0

        jax.lax.fori_loop(0, _B, loop, 0)

        def exact_b():
            def bloop(j, acc):
                blk = dv_ref[j]
                return acc + jnp.sum(jnp.where(blk > t, blk, 0.0))
            return jax.lax.fori_loop(0, _B, bloop, 0.0)

        b = jax.lax.cond(t < 0.05, exact_b, lambda: 0.0)
        total = jnp.sum(s_ref[...]) + (jnp.sum(w_ref[...]) - _EPS * b) / denom
        out_ref[0] = total / jnp.sum(c_ref[...])


def kernel(pred, gt):
    def spec(k):
        return pl.BlockSpec((1, 1, _H, _W), lambda i, k=k: (_K * i + k, 0, 0, 0))

    out = pl.pallas_call(
        _berhu_body,
        grid=(_NSTEPS,),
        in_specs=[spec(k) for k in range(_K)] * 2,
        out_specs=pl.BlockSpec(memory_space=pltpu.SMEM),
        out_shape=jax.ShapeDtypeStruct((1,), jnp.float32),
        scratch_shapes=[
            pltpu.VMEM((_B, _H, _W), jnp.float32),
            pltpu.VMEM((_H, _W), jnp.float32),
            pltpu.VMEM((_H, _W), jnp.float32),
            pltpu.VMEM((_H, _W), jnp.float32),
            pltpu.VMEM((_H, _W), jnp.float32),
        ],
        compiler_params=pltpu.CompilerParams(
            vmem_limit_bytes=58 * 1024 * 1024,
        ),
    )(pred, pred, pred, pred, gt, gt, gt, gt)
    return out[0]


# speedup vs baseline: 1.1771x; 1.0148x over previous
import jax
import jax.numpy as jnp
from jax.experimental import pallas as pl
from jax.experimental.pallas import tpu as pltpu

_SCALE = 0.5
_EPS = 1e-05

_B = 32
_H = 512
_W = 512
_HH = _H // 2
_K = 8
_BPS = _K // 2
_NSTEPS = _B // _BPS


def _berhu_body(*refs):
    preds = refs[:_K]
    gts = refs[_K:2 * _K]
    out_ref = refs[2 * _K]
    dv_ref, s_ref, m_ref, c_ref, w_ref = refs[2 * _K + 1:]
    i = pl.program_id(0)

    @pl.when(i == 0)
    def _init():
        s_ref[...] = jnp.zeros_like(s_ref)
        m_ref[...] = jnp.zeros_like(m_ref)
        c_ref[...] = jnp.zeros_like(c_ref)

    s = s_ref[...]
    m = m_ref[...]
    c = c_ref[...]
    for k in range(_K):
        p = preds[k][0, 0]
        g = gts[k][0, 0]
        valid = g > _EPS
        dv = jnp.where(valid, jnp.abs(p - g), 0.0)
        dv_ref[_BPS * i + k // 2, (k % 2) * _HH:(k % 2 + 1) * _HH, :] = dv
        s = s + dv
        m = jnp.maximum(m, dv)
        c = c + jnp.where(valid, 1.0, 0.0)
    s_ref[...] = s
    m_ref[...] = m
    c_ref[...] = c

    @pl.when(i == _NSTEPS - 1)
    def _finish():
        t = _SCALE * jnp.max(m_ref[...])
        denom = 2.0 * t + _EPS
        w_ref[...] = jnp.zeros_like(w_ref)

        def loop(j, _):
            q = jnp.maximum(dv_ref[j] - t, 0.0)
            w_ref[...] = w_ref[...] + q * q
            return 0

        jax.lax.fori_loop(0, _B, loop, 0)

        def exact_b():
            def bloop(j, acc):
                blk = dv_ref[j]
                return acc + jnp.sum(jnp.where(blk > t, blk, 0.0))
            return jax.lax.fori_loop(0, _B, bloop, 0.0)

        b = jax.lax.cond(t < 0.05, exact_b, lambda: 0.0)
        total = jnp.sum(s_ref[...]) + (jnp.sum(w_ref[...]) - _EPS * b) / denom
        out_ref[0] = total / jnp.sum(c_ref[...])


def kernel(pred, gt):
    def spec(k):
        return pl.BlockSpec(
            (1, 1, _HH, _W),
            lambda i, k=k: (_BPS * i + k // 2, 0, k % 2, 0))

    out = pl.pallas_call(
        _berhu_body,
        grid=(_NSTEPS,),
        in_specs=[spec(k) for k in range(_K)] * 2,
        out_specs=pl.BlockSpec(memory_space=pltpu.SMEM),
        out_shape=jax.ShapeDtypeStruct((1,), jnp.float32),
        scratch_shapes=[
            pltpu.VMEM((_B, _H, _W), jnp.float32),
            pltpu.VMEM((_HH, _W), jnp.float32),
            pltpu.VMEM((_HH, _W), jnp.float32),
            pltpu.VMEM((_HH, _W), jnp.float32),
            pltpu.VMEM((_H, _W), jnp.float32),
        ],
        compiler_params=pltpu.CompilerParams(
            vmem_limit_bytes=58 * 1024 * 1024,
        ),
    )(*([pred] * _K + [gt] * _K))
    return out[0]


# speedup vs baseline: 1.1882x; 1.0094x over previous
import jax
import jax.numpy as jnp
from jax.experimental import pallas as pl
from jax.experimental.pallas import tpu as pltpu

_SCALE = 0.5
_EPS = 1e-05

_B = 32
_H = 512
_W = 512
_HH = _H // 2
_K = 8
_BPS = _K // 2
_NSTEPS = _B // _BPS


def _berhu_body(*refs):
    preds = refs[:_K]
    gts = refs[_K:2 * _K]
    out_ref = refs[2 * _K]
    dv_ref, s_ref, m_ref, c_ref, w_ref = refs[2 * _K + 1:]
    i = pl.program_id(0)

    @pl.when(i == 0)
    def _init():
        s_ref[...] = jnp.zeros_like(s_ref)
        m_ref[...] = jnp.zeros_like(m_ref)
        c_ref[...] = jnp.zeros_like(c_ref)

    s = s_ref[...]
    m = m_ref[...]
    c = c_ref[...]
    for k in range(_K):
        p = preds[k][0, 0]
        g = gts[k][0, 0]
        valid = g > _EPS
        dv = jnp.where(valid, jnp.abs(p - g), 0.0)
        dv_ref[_BPS * i + k // 2, (k % 2) * _HH:(k % 2 + 1) * _HH, :] = (
            dv.astype(jnp.bfloat16))
        s = s + dv
        m = jnp.maximum(m, dv)
        c = c + jnp.where(valid, 1.0, 0.0)
    s_ref[...] = s
    m_ref[...] = m
    c_ref[...] = c

    @pl.when(i == _NSTEPS - 1)
    def _finish():
        t = _SCALE * jnp.max(m_ref[...])
        denom = 2.0 * t + _EPS
        t_bf = t.astype(jnp.bfloat16)
        w_ref[...] = jnp.zeros_like(w_ref)

        def loop(j, _):
            q = jnp.maximum(dv_ref[j] - t_bf, jnp.bfloat16(0.0))
            w_ref[...] = w_ref[...] + (q * q).astype(jnp.float32)
            return 0

        jax.lax.fori_loop(0, _B, loop, 0)

        def exact_b():
            def bloop(j, acc):
                blk = dv_ref[j].astype(jnp.float32)
                return acc + jnp.sum(jnp.where(blk > t, blk, 0.0))
            return jax.lax.fori_loop(0, _B, bloop, 0.0)

        b = jax.lax.cond(t < 0.05, exact_b, lambda: 0.0)
        total = jnp.sum(s_ref[...]) + (jnp.sum(w_ref[...]) - _EPS * b) / denom
        out_ref[0] = total / jnp.sum(c_ref[...])


def kernel(pred, gt):
    def spec(k):
        return pl.BlockSpec(
            (1, 1, _HH, _W),
            lambda i, k=k: (_BPS * i + k // 2, 0, k % 2, 0))

    out = pl.pallas_call(
        _berhu_body,
        grid=(_NSTEPS,),
        in_specs=[spec(k) for k in range(_K)] * 2,
        out_specs=pl.BlockSpec(memory_space=pltpu.SMEM),
        out_shape=jax.ShapeDtypeStruct((1,), jnp.float32),
        scratch_shapes=[
            pltpu.VMEM((_B, _H, _W), jnp.bfloat16),
            pltpu.VMEM((_HH, _W), jnp.float32),
            pltpu.VMEM((_HH, _W), jnp.float32),
            pltpu.VMEM((_HH, _W), jnp.float32),
            pltpu.VMEM((_H, _W), jnp.float32),
        ],
        compiler_params=pltpu.CompilerParams(
            vmem_limit_bytes=58 * 1024 * 1024,
        ),
    )(*([pred] * _K + [gt] * _K))
    return out[0]


# speedup vs baseline: 1.2110x; 1.0192x over previous
import jax
import jax.numpy as jnp
from jax.experimental import pallas as pl
from jax.experimental.pallas import tpu as pltpu

_SCALE = 0.5
_EPS = 1e-05

_B = 32
_H = 512
_W = 512
_HH = _H // 4
_K = 16
_BPS = _K // 4
_NSTEPS = _B // _BPS


def _berhu_body(*refs):
    preds = refs[:_K]
    gts = refs[_K:2 * _K]
    out_ref = refs[2 * _K]
    dv_ref, s_ref, m_ref, c_ref, w_ref = refs[2 * _K + 1:]
    i = pl.program_id(0)

    @pl.when(i == 0)
    def _init():
        s_ref[...] = jnp.zeros_like(s_ref)
        m_ref[...] = jnp.zeros_like(m_ref)
        c_ref[...] = jnp.zeros_like(c_ref)

    s = s_ref[...]
    m = m_ref[...]
    c = c_ref[...]
    for k in range(_K):
        p = preds[k][0, 0]
        g = gts[k][0, 0]
        valid = g > _EPS
        dv = jnp.where(valid, jnp.abs(p - g), 0.0)
        dv_ref[_BPS * i + k // 4, (k % 4) * _HH:(k % 4 + 1) * _HH, :] = (
            dv.astype(jnp.bfloat16))
        s = s + dv
        m = jnp.maximum(m, dv)
        c = c + jnp.where(valid, 1.0, 0.0)
    s_ref[...] = s
    m_ref[...] = m
    c_ref[...] = c

    @pl.when(i == _NSTEPS - 1)
    def _finish():
        t = _SCALE * jnp.max(m_ref[...])
        denom = 2.0 * t + _EPS
        t_bf = t.astype(jnp.bfloat16)
        w_ref[...] = jnp.zeros_like(w_ref)

        def loop(j, _):
            q = jnp.maximum(dv_ref[j] - t_bf, jnp.bfloat16(0.0))
            w_ref[...] = w_ref[...] + (q * q).astype(jnp.float32)
            return 0

        jax.lax.fori_loop(0, _B, loop, 0)

        def exact_b():
            def bloop(j, acc):
                blk = dv_ref[j].astype(jnp.float32)
                return acc + jnp.sum(jnp.where(blk > t, blk, 0.0))
            return jax.lax.fori_loop(0, _B, bloop, 0.0)

        b = jax.lax.cond(t < 0.05, exact_b, lambda: 0.0)
        total = jnp.sum(s_ref[...]) + (jnp.sum(w_ref[...]) - _EPS * b) / denom
        out_ref[0] = total / jnp.sum(c_ref[...])


def kernel(pred, gt):
    def spec(k):
        return pl.BlockSpec(
            (1, 1, _HH, _W),
            lambda i, k=k: (_BPS * i + k // 4, 0, k % 4, 0))

    out = pl.pallas_call(
        _berhu_body,
        grid=(_NSTEPS,),
        in_specs=[spec(k) for k in range(_K)] * 2,
        out_specs=pl.BlockSpec(memory_space=pltpu.SMEM),
        out_shape=jax.ShapeDtypeStruct((1,), jnp.float32),
        scratch_shapes=[
            pltpu.VMEM((_B, _H, _W), jnp.bfloat16),
            pltpu.VMEM((_HH, _W), jnp.float32),
            pltpu.VMEM((_HH, _W), jnp.float32),
            pltpu.VMEM((_HH, _W), jnp.float32),
            pltpu.VMEM((_H, _W), jnp.float32),
        ],
        compiler_params=pltpu.CompilerParams(
            vmem_limit_bytes=58 * 1024 * 1024,
        ),
    )(*([pred] * _K + [gt] * _K))
    return out[0]


# speedup vs baseline: 1.2308x; 1.0163x over previous
import jax
import jax.numpy as jnp
from jax.experimental import pallas as pl
from jax.experimental.pallas import tpu as pltpu

_SCALE = 0.5
_EPS = 1e-05

_B = 32
_H = 512
_W = 512
_HH = _H // 4
_K = 16
_BPS = _K // 4
_NSTEPS = _B // _BPS


def _berhu_body(*refs):
    preds = refs[:_K]
    gts = refs[_K:2 * _K]
    out_ref = refs[2 * _K]
    dv_ref, s_ref, m_ref, c_ref = refs[2 * _K + 1:]
    i = pl.program_id(0)

    @pl.when(i == 0)
    def _init():
        s_ref[...] = jnp.zeros_like(s_ref)
        m_ref[...] = jnp.zeros_like(m_ref)
        c_ref[...] = jnp.zeros_like(c_ref)

    s = s_ref[...]
    m = m_ref[...]
    c = c_ref[...]
    for k in range(_K):
        p = preds[k][0, 0]
        g = gts[k][0, 0]
        valid = g > _EPS
        dv = jnp.where(valid, jnp.abs(p - g), 0.0)
        dv_ref[_BPS * i + k // 4, (k % 4) * _HH:(k % 4 + 1) * _HH, :] = (
            dv.astype(jnp.bfloat16))
        s = s + dv
        m = jnp.maximum(m, dv)
        c = c + jnp.where(valid, 1.0, 0.0)
    s_ref[...] = s
    m_ref[...] = m
    c_ref[...] = c

    @pl.when(i == _NSTEPS - 1)
    def _finish():
        t = _SCALE * jnp.max(m_ref[...])
        denom = 2.0 * t + _EPS
        t_bf = t.astype(jnp.bfloat16)

        def loop(j, acc):
            q = jnp.maximum(dv_ref[j] - t_bf, jnp.bfloat16(0.0))
            x = q * q
            x = x[:256] + x[256:]
            x = x[:128] + x[128:]
            x = x[:64] + x[64:]
            x = x[:32] + x[32:]
            x = x[:16] + x[16:]
            return acc + x.astype(jnp.float32)

        w = jax.lax.fori_loop(0, _B, loop,
                              jnp.zeros((16, _W), jnp.float32))

        def exact_b():
            def bloop(j, acc):
                blk = dv_ref[j].astype(jnp.float32)
                return acc + jnp.sum(jnp.where(blk > t, blk, 0.0))
            return jax.lax.fori_loop(0, _B, bloop, 0.0)

        b = jax.lax.cond(t < 0.05, exact_b, lambda: 0.0)
        total = jnp.sum(s_ref[...]) + (jnp.sum(w) - _EPS * b) / denom
        out_ref[0] = total / jnp.sum(c_ref[...])


def kernel(pred, gt):
    def spec(k):
        return pl.BlockSpec(
            (1, 1, _HH, _W),
            lambda i, k=k: (_BPS * i + k // 4, 0, k % 4, 0))

    out = pl.pallas_call(
        _berhu_body,
        grid=(_NSTEPS,),
        in_specs=[spec(k) for k in range(_K)] * 2,
        out_specs=pl.BlockSpec(memory_space=pltpu.SMEM),
        out_shape=jax.ShapeDtypeStruct((1,), jnp.float32),
        scratch_shapes=[
            pltpu.VMEM((_B, _H, _W), jnp.bfloat16),
            pltpu.VMEM((_HH, _W), jnp.float32),
            pltpu.VMEM((_HH, _W), jnp.float32),
            pltpu.VMEM((_HH, _W), jnp.float32),
        ],
        compiler_params=pltpu.CompilerParams(
            vmem_limit_bytes=58 * 1024 * 1024,
        ),
    )(*([pred] * _K + [gt] * _K))
    return out[0]


# speedup vs baseline: 1.2551x; 1.0198x over previous
import jax
import jax.numpy as jnp
from jax.experimental import pallas as pl
from jax.experimental.pallas import tpu as pltpu

_SCALE = 0.5
_EPS = 1e-05

_B = 32
_H = 512
_W = 512
_HH = _H // 4
_K = 16
_BPS = _K // 4
_NSTEPS = _B // _BPS


def _berhu_body(*refs):
    preds = refs[:_K]
    gts = refs[_K:2 * _K]
    out_ref = refs[2 * _K]
    dv_ref, s_ref, m_ref, c_ref = refs[2 * _K + 1:]
    i = pl.program_id(0)

    @pl.when(i == 0)
    def _init():
        s_ref[...] = jnp.zeros_like(s_ref)
        m_ref[...] = jnp.zeros_like(m_ref)
        c_ref[...] = jnp.zeros_like(c_ref)

    ones = jnp.ones((8, _HH), jnp.bfloat16)
    s = s_ref[...]
    m = m_ref[...]
    c = c_ref[...]
    for k in range(_K):
        p = preds[k][0, 0]
        g = gts[k][0, 0]
        valid = g > _EPS
        dv = jnp.where(valid, jnp.abs(p - g), 0.0)
        dvb = dv.astype(jnp.bfloat16)
        dv_ref[_BPS * i + k // 4, (k % 4) * _HH:(k % 4 + 1) * _HH, :] = dvb
        s = s + jax.lax.dot(ones, dvb,
                            preferred_element_type=jnp.float32)
        c = c + jnp.where(valid, 1.0, 0.0)
        m = jnp.maximum(m, dv)
    s_ref[...] = s
    m_ref[...] = m
    c_ref[...] = c

    @pl.when(i == _NSTEPS - 1)
    def _finish():
        t = _SCALE * jnp.max(m_ref[...])
        denom = 2.0 * t + _EPS
        t_bf = t.astype(jnp.bfloat16)

        def loop(j, acc):
            q = jnp.maximum(dv_ref[j] - t_bf, jnp.bfloat16(0.0))
            x = q * q
            x = x[:256] + x[256:]
            x = x[:128] + x[128:]
            x = x[:64] + x[64:]
            x = x[:32] + x[32:]
            x = x[:16] + x[16:]
            return acc + x.astype(jnp.float32)

        w = jax.lax.fori_loop(0, _B, loop,
                              jnp.zeros((16, _W), jnp.float32))

        def exact_b():
            def bloop(j, acc):
                blk = dv_ref[j].astype(jnp.float32)
                return acc + jnp.sum(jnp.where(blk > t, blk, 0.0))
            return jax.lax.fori_loop(0, _B, bloop, 0.0)

        b = jax.lax.cond(t < 0.05, exact_b, lambda: 0.0)
        total = 0.125 * jnp.sum(s_ref[...]) + (jnp.sum(w) - _EPS * b) / denom
        out_ref[0] = total / jnp.sum(c_ref[...])


def kernel(pred, gt):
    def spec(k):
        return pl.BlockSpec(
            (1, 1, _HH, _W),
            lambda i, k=k: (_BPS * i + k // 4, 0, k % 4, 0))

    out = pl.pallas_call(
        _berhu_body,
        grid=(_NSTEPS,),
        in_specs=[spec(k) for k in range(_K)] * 2,
        out_specs=pl.BlockSpec(memory_space=pltpu.SMEM),
        out_shape=jax.ShapeDtypeStruct((1,), jnp.float32),
        scratch_shapes=[
            pltpu.VMEM((_B, _H, _W), jnp.bfloat16),
            pltpu.VMEM((8, _W), jnp.float32),
            pltpu.VMEM((_HH, _W), jnp.float32),
            pltpu.VMEM((_HH, _W), jnp.float32),
        ],
        compiler_params=pltpu.CompilerParams(
            vmem_limit_bytes=58 * 1024 * 1024,
        ),
    )(*([pred] * _K + [gt] * _K))
    return out[0]


# speedup vs baseline: 1.2696x; 1.0116x over previous
import jax
import jax.numpy as jnp
from jax.experimental import pallas as pl
from jax.experimental.pallas import tpu as pltpu

_SCALE = 0.5
_EPS = 1e-05

_B = 32
_H = 512
_W = 512
_HH = _H // 4
_K = 16
_BPS = _K // 4
_NSTEPS = _B // _BPS


def _berhu_body(*refs):
    preds = refs[:_K]
    gts = refs[_K:2 * _K]
    out_ref = refs[2 * _K]
    dv_ref, s_ref, m_ref, c_ref = refs[2 * _K + 1:]
    i = pl.program_id(0)

    @pl.when(i == 0)
    def _init():
        s_ref[...] = jnp.zeros_like(s_ref)
        m_ref[...] = jnp.zeros_like(m_ref)
        c_ref[...] = jnp.zeros_like(c_ref)

    ones = jnp.ones((8, _HH), jnp.bfloat16)
    s = s_ref[...]
    m = m_ref[...]
    c = c_ref[...]
    for k in range(_K):
        p = preds[k][0, 0]
        g = gts[k][0, 0]
        valid = g > _EPS
        dv = jnp.where(valid, jnp.abs(p - g), 0.0)
        dvb = dv.astype(jnp.bfloat16)
        dv_ref[_BPS * i + k // 4, (k % 4) * _HH:(k % 4 + 1) * _HH, :] = dvb
        s = s + jax.lax.dot(ones, dvb,
                            preferred_element_type=jnp.float32)
        c = c + jnp.where(valid, 1.0, 0.0)
        m = jnp.maximum(m, dvb)
    s_ref[...] = s
    m_ref[...] = m
    c_ref[...] = c

    @pl.when(i == _NSTEPS - 1)
    def _finish():
        t = _SCALE * jnp.max(m_ref[...].astype(jnp.float32))
        denom = 2.0 * t + _EPS
        t_bf = t.astype(jnp.bfloat16)

        def loop(j, acc):
            q0 = jnp.maximum(dv_ref[2 * j] - t_bf, jnp.bfloat16(0.0))
            q1 = jnp.maximum(dv_ref[2 * j + 1] - t_bf, jnp.bfloat16(0.0))
            x = q0 * q0 + q1 * q1
            x = x[:256] + x[256:]
            x = x[:128] + x[128:]
            x = x[:64] + x[64:]
            x = x[:32] + x[32:]
            x = x[:16] + x[16:]
            return acc + x.astype(jnp.float32)

        w = jax.lax.fori_loop(0, _B // 2, loop,
                              jnp.zeros((16, _W), jnp.float32))

        def exact_b():
            def bloop(j, acc):
                blk = dv_ref[j].astype(jnp.float32)
                return acc + jnp.sum(jnp.where(blk > t, blk, 0.0))
            return jax.lax.fori_loop(0, _B, bloop, 0.0)

        b = jax.lax.cond(t < 0.05, exact_b, lambda: 0.0)
        total = 0.125 * jnp.sum(s_ref[...]) + (jnp.sum(w) - _EPS * b) / denom
        out_ref[0] = total / jnp.sum(c_ref[...])


def kernel(pred, gt):
    def spec(k):
        return pl.BlockSpec(
            (1, 1, _HH, _W),
            lambda i, k=k: (_BPS * i + k // 4, 0, k % 4, 0))

    out = pl.pallas_call(
        _berhu_body,
        grid=(_NSTEPS,),
        in_specs=[spec(k) for k in range(_K)] * 2,
        out_specs=pl.BlockSpec(memory_space=pltpu.SMEM),
        out_shape=jax.ShapeDtypeStruct((1,), jnp.float32),
        scratch_shapes=[
            pltpu.VMEM((_B, _H, _W), jnp.bfloat16),
            pltpu.VMEM((8, _W), jnp.float32),
            pltpu.VMEM((_HH, _W), jnp.bfloat16),
            pltpu.VMEM((_HH, _W), jnp.float32),
        ],
        compiler_params=pltpu.CompilerParams(
            vmem_limit_bytes=58 * 1024 * 1024,
        ),
    )(*([pred] * _K + [gt] * _K))
    return out[0]
